# baseline (speedup 1.0000x reference)
import jax
import jax.numpy as jnp
from jax import lax
from jax.experimental import pallas as pl
from jax.experimental.pallas import tpu as pltpu
from jax.experimental.pallas import tpu_sc as plsc

N_NODES = 10000
N_EDGES = 320000
D = 128
OUT = 128
BOUND = 10.0

_NC = 2
_NS = 16
_NW = _NC * _NS
_E_TILE = N_EDGES // _NW
_CHUNK = 80
_NCHUNK = _E_TILE // _CHUNK
_NB = 3
_ROWS_TILE = N_NODES // _NS


def _sc_agg_body(x_hbm, ei_hbm, dst_hbm, out_hbm,
                 src_v, dst_v, rows, agg_sh, gsem, ssem):
    c = lax.axis_index("c")
    s = lax.axis_index("s")
    tile = c * _NS + s

    pltpu.sync_copy(ei_hbm.at[0, pl.ds(tile * _E_TILE, _E_TILE)], src_v)
    pltpu.sync_copy(dst_hbm.at[tile], dst_v)
    pltpu.async_copy(x_hbm.at[src_v.at[pl.ds(0, _CHUNK)]], rows.at[0], gsem[0])

    zvec = jnp.zeros((16,), jnp.float32)

    def _zero_elem(k, _):
        i = k // (D // 16)
        j = k % (D // 16)
        rows[1, i, pl.ds(j * 16, 16)] = zvec
        return 0

    lax.fori_loop(0, _CHUNK * (D // 16), _zero_elem, 0)
    r0 = s * _ROWS_TILE
    for k in range(_ROWS_TILE // _CHUNK):
        pltpu.sync_copy(rows.at[1], agg_sh.at[pl.ds(r0 + k * _CHUNK, _CHUNK)])
    _rem = _ROWS_TILE % _CHUNK
    pltpu.sync_copy(rows.at[1, pl.ds(0, _rem)],
                    agg_sh.at[pl.ds(r0 + _ROWS_TILE - _rem, _rem)])
    plsc.subcore_barrier()

    def _src_sl(i):
        return src_v.at[pl.ds(i * _CHUNK, _CHUNK)]

    def _dst_sl(i):
        return dst_v.at[i]

    pltpu.async_copy(x_hbm.at[_src_sl(1)], rows.at[1], gsem[1])
    pltpu.async_copy(x_hbm.at[_src_sl(2)], rows.at[2], gsem[2])

    def _edge_trip(k, _):
        i = 3 * k
        for b in range(_NB):
            pltpu.make_async_copy(x_hbm.at[_src_sl(i + b)],
                                  rows.at[b], gsem[b]).wait()
            pltpu.sync_copy(rows.at[b], agg_sh.at[_dst_sl(i + b)], add=True)
            pltpu.async_copy(x_hbm.at[_src_sl(i + b + 3)],
                             rows.at[b], gsem[b])
        return 0

    lax.fori_loop(0, _NCHUNK // 3 - 1, _edge_trip, 0)
    i = _NCHUNK - 5
    for b in range(_NB):
        pltpu.make_async_copy(x_hbm.at[_src_sl(i + b)],
                              rows.at[b], gsem[b]).wait()
        pltpu.sync_copy(rows.at[b], agg_sh.at[_dst_sl(i + b)], add=True)
    for b in range(2):
        pltpu.async_copy(x_hbm.at[_src_sl(i + 3 + b)], rows.at[b], gsem[b])
    for b in range(2):
        pltpu.make_async_copy(x_hbm.at[_src_sl(i + 3 + b)],
                              rows.at[b], gsem[b]).wait()
        pltpu.sync_copy(rows.at[b], agg_sh.at[_dst_sl(i + 3 + b)], add=True)
    plsc.subcore_barrier()

    pltpu.sync_copy(agg_sh.at[pl.ds(r0, _ROWS_TILE)],
                    out_hbm.at[c, pl.ds(r0, _ROWS_TILE)])


import functools


@functools.cache
def _get_sc_agg():
    return pl.kernel(
        _sc_agg_body,
        out_type=jax.ShapeDtypeStruct((_NC, N_NODES, D), jnp.float32),
        mesh=plsc.VectorSubcoreMesh(core_axis_name="c", subcore_axis_name="s",
                                    num_cores=_NC, num_subcores=_NS),
        compiler_params=pltpu.CompilerParams(use_tc_tiling_on_sc=False),
        scratch_types=[
            pltpu.VMEM((_E_TILE,), jnp.int32),
            pltpu.VMEM((_NCHUNK, _CHUNK), jnp.int32),
            pltpu.VMEM((_NB, _CHUNK, D), jnp.float32),
            pltpu.VMEM_SHARED((N_NODES, D), jnp.float32),
            [pltpu.SemaphoreType.DMA] * _NB,
            [pltpu.SemaphoreType.DMA] * _NB,
        ],
    )


_BLK = 2000
_NBLK = N_NODES // _BLK


def _tc_body(x_ref, p_ref, w_ref, b_ref, out_ref, loss_ref):
    a = jnp.clip(p_ref[0] + p_ref[1], 0.0, BOUND)
    x1 = (jnp.dot(x_ref[...], w_ref[0:D], preferred_element_type=jnp.float32)
          + jnp.dot(a, w_ref[D:2 * D], preferred_element_type=jnp.float32)
          + b_ref[...])
    m = jnp.max(x1, axis=-1, keepdims=True)
    col = lax.broadcasted_iota(jnp.int32, x1.shape, 1)
    idx = jnp.min(jnp.where(x1 == m, col, OUT), axis=-1, keepdims=True)
    x2 = (col == idx).astype(jnp.float32)
    out_ref[...] = x2

    @pl.when(pl.program_id(0) == 0)
    def _init():
        loss_ref[0, 0] = 0.0

    loss_ref[0, 0] += jnp.sum((x2 - x1) ** 2)


_tc = pl.pallas_call(
    _tc_body,
    grid=(_NBLK,),
    in_specs=[
        pl.BlockSpec((_BLK, D), lambda i: (i, 0)),
        pl.BlockSpec((_NC, _BLK, D), lambda i: (0, i, 0)),
        pl.BlockSpec((2 * D, OUT), lambda i: (0, 0)),
        pl.BlockSpec((1, OUT), lambda i: (0, 0)),
    ],
    out_specs=[
        pl.BlockSpec((_BLK, OUT), lambda i: (i, 0)),
        pl.BlockSpec((1, 1), lambda i: (0, 0), memory_space=pltpu.SMEM),
    ],
    out_shape=[
        jax.ShapeDtypeStruct((N_NODES, OUT), jnp.float32),
        jax.ShapeDtypeStruct((1, 1), jnp.float32),
    ],
)


def kernel(x, edge_index, W, b):
    dst = edge_index[1].reshape(_NW, _NCHUNK, _CHUNK)
    partials = _get_sc_agg()(x, edge_index, dst)
    x2, loss = _tc(x, partials, W, b.reshape(1, OUT))
    return x2, loss[0, 0] / (N_NODES * OUT)


# speedup vs baseline: 1.1095x; 1.1095x over previous
import jax
import jax.numpy as jnp
from jax import lax
from jax.experimental import pallas as pl
from jax.experimental.pallas import tpu as pltpu
from jax.experimental.pallas import tpu_sc as plsc

N_NODES = 10000
N_EDGES = 320000
D = 128
OUT = 128
BOUND = 10.0

_NC = 2
_NS = 16
_NW = _NC * _NS
_E_TILE = N_EDGES // _NW
_CHUNK = 80
_NCHUNK = _E_TILE // _CHUNK
_NB = 3
_ROWS_TILE = N_NODES // _NS


def _sc_agg_body(x_hbm, ei_hbm, out_hbm,
                 src_v, dst_v, rows, agg_sh, gsem, ssem):
    c = lax.axis_index("c")
    s = lax.axis_index("s")
    tile = c * _NS + s

    pltpu.sync_copy(ei_hbm.at[0, tile], src_v)
    pltpu.sync_copy(ei_hbm.at[1, tile], dst_v)
    pltpu.async_copy(x_hbm.at[src_v.at[0]], rows.at[0], gsem[0])

    zvec = jnp.zeros((16,), jnp.float32)

    def _zero_elem(k, _):
        i = k // (D // 16)
        j = k % (D // 16)
        rows[1, i, pl.ds(j * 16, 16)] = zvec
        return 0

    lax.fori_loop(0, _CHUNK * (D // 16), _zero_elem, 0)
    r0 = s * _ROWS_TILE
    for k in range(_ROWS_TILE // _CHUNK):
        pltpu.sync_copy(rows.at[1], agg_sh.at[pl.ds(r0 + k * _CHUNK, _CHUNK)])
    _rem = _ROWS_TILE % _CHUNK
    pltpu.sync_copy(rows.at[1, pl.ds(0, _rem)],
                    agg_sh.at[pl.ds(r0 + _ROWS_TILE - _rem, _rem)])
    plsc.subcore_barrier()

    pltpu.async_copy(x_hbm.at[src_v.at[1]], rows.at[1], gsem[1])
    pltpu.async_copy(x_hbm.at[src_v.at[2]], rows.at[2], gsem[2])

    def _edge_trip(k, _):
        i = 3 * k
        for b in range(_NB):
            pltpu.make_async_copy(x_hbm.at[src_v.at[i + b]],
                                  rows.at[b], gsem[b]).wait()
            pltpu.sync_copy(rows.at[b], agg_sh.at[dst_v.at[i + b]], add=True)
            pltpu.async_copy(x_hbm.at[src_v.at[i + b + 3]],
                             rows.at[b], gsem[b])
        return 0

    lax.fori_loop(0, _NCHUNK // 3 - 1, _edge_trip, 0)
    i = _NCHUNK - 5
    for b in range(_NB):
        pltpu.make_async_copy(x_hbm.at[src_v.at[i + b]],
                              rows.at[b], gsem[b]).wait()
        pltpu.sync_copy(rows.at[b], agg_sh.at[dst_v.at[i + b]], add=True)
    for b in range(2):
        pltpu.async_copy(x_hbm.at[src_v.at[i + 3 + b]], rows.at[b], gsem[b])
    for b in range(2):
        pltpu.make_async_copy(x_hbm.at[src_v.at[i + 3 + b]],
                              rows.at[b], gsem[b]).wait()
        pltpu.sync_copy(rows.at[b], agg_sh.at[dst_v.at[i + 3 + b]], add=True)
    plsc.subcore_barrier()

    pltpu.sync_copy(agg_sh.at[pl.ds(r0, _ROWS_TILE)],
                    out_hbm.at[c, pl.ds(r0, _ROWS_TILE)])


import functools


@functools.cache
def _get_sc_agg():
    return pl.kernel(
        _sc_agg_body,
        out_type=jax.ShapeDtypeStruct((_NC, N_NODES, D), jnp.float32),
        mesh=plsc.VectorSubcoreMesh(core_axis_name="c", subcore_axis_name="s",
                                    num_cores=_NC, num_subcores=_NS),
        compiler_params=pltpu.CompilerParams(use_tc_tiling_on_sc=False),
        scratch_types=[
            pltpu.VMEM((_NCHUNK, _CHUNK), jnp.int32),
            pltpu.VMEM((_NCHUNK, _CHUNK), jnp.int32),
            pltpu.VMEM((_NB, _CHUNK, D), jnp.float32),
            pltpu.VMEM_SHARED((N_NODES, D), jnp.float32),
            [pltpu.SemaphoreType.DMA] * _NB,
            [pltpu.SemaphoreType.DMA] * _NB,
        ],
    )


_BLK = 2000
_NBLK = N_NODES // _BLK


def _tc_body(x_ref, p_ref, w_ref, b_ref, out_ref, loss_ref):
    a = jnp.clip(p_ref[0] + p_ref[1], 0.0, BOUND)
    x1 = (jnp.dot(x_ref[...], w_ref[0:D], preferred_element_type=jnp.float32)
          + jnp.dot(a, w_ref[D:2 * D], preferred_element_type=jnp.float32)
          + b_ref[...])
    m = jnp.max(x1, axis=-1, keepdims=True)
    col = lax.broadcasted_iota(jnp.int32, x1.shape, 1)
    idx = jnp.min(jnp.where(x1 == m, col, OUT), axis=-1, keepdims=True)
    x2 = (col == idx).astype(jnp.float32)
    out_ref[...] = x2

    @pl.when(pl.program_id(0) == 0)
    def _init():
        loss_ref[0, 0] = 0.0

    loss_ref[0, 0] += jnp.sum((x2 - x1) ** 2)


_tc = pl.pallas_call(
    _tc_body,
    grid=(_NBLK,),
    in_specs=[
        pl.BlockSpec((_BLK, D), lambda i: (i, 0)),
        pl.BlockSpec((_NC, _BLK, D), lambda i: (0, i, 0)),
        pl.BlockSpec((2 * D, OUT), lambda i: (0, 0)),
        pl.BlockSpec((1, OUT), lambda i: (0, 0)),
    ],
    out_specs=[
        pl.BlockSpec((_BLK, OUT), lambda i: (i, 0)),
        pl.BlockSpec((1, 1), lambda i: (0, 0), memory_space=pltpu.SMEM),
    ],
    out_shape=[
        jax.ShapeDtypeStruct((N_NODES, OUT), jnp.float32),
        jax.ShapeDtypeStruct((1, 1), jnp.float32),
    ],
)


def kernel(x, edge_index, W, b):
    ei = edge_index.reshape(2, _NW, _NCHUNK, _CHUNK)
    partials = _get_sc_agg()(x, ei)
    x2, loss = _tc(x, partials, W, b.reshape(1, OUT))
    return x2, loss[0, 0] / (N_NODES * OUT)
